# Initial kernel scaffold; baseline (speedup 1.0000x reference)
import jax
import jax.numpy as jnp
from jax.experimental import pallas as pl


def kernel(x, edge_index, W_self0, W_neigh0, b0, W_self1, W_neigh1, b1, W_self2, W_neigh2, b2):
    raise NotImplementedError("write your pallas kernel here")


# speedup vs baseline: 5.4104x; 5.4104x over previous
import functools

import jax
import jax.numpy as jnp
from jax import lax
from jax.experimental import pallas as pl
from jax.experimental.pallas import tpu as pltpu
from jax.experimental.pallas import tpu_sc as plsc

N = 10000
N_PAD = 10240
E = 320000
D_IN = 128
D_HID = 128
D_OUT = 40

NC = 2
NS = 16
NW = NC * NS
CHUNK = 128
NCHUNKS = E // CHUNK
CHUNKS_PER_W = -(-NCHUNKS // NW)
ROWS_PER_TILE = N_PAD // NS
DEG_W = 128

_sc_mesh = plsc.VectorSubcoreMesh(core_axis_name="c", subcore_axis_name="s")


def _make_sc_segsum(d):
  scratch = [
      pltpu.VMEM((CHUNK,), jnp.int32),
      pltpu.VMEM((CHUNK,), jnp.int32),
      pltpu.VMEM((CHUNK, d), jnp.float32),
      pltpu.VMEM((CHUNK, d), jnp.float32),
      pltpu.VMEM_SHARED((N_PAD, d), jnp.float32),
      pltpu.SemaphoreType.DMA,
  ]

  def body(src_hbm, dst_hbm, z_hbm, zrow_hbm, agg_out,
           src_v, dst_v, rows_v, stage_v, agg_sh, sem):
    c = lax.axis_index("c")
    s = lax.axis_index("s")
    w = c * NS + s
    row0 = s * ROWS_PER_TILE
    nsub = ROWS_PER_TILE // CHUNK
    pltpu.sync_copy(zrow_hbm, stage_v)
    for k in range(nsub):
      pltpu.sync_copy(stage_v, agg_sh.at[pl.ds(row0 + k * CHUNK, CHUNK)])
    plsc.subcore_barrier()

    def step(jj, carry):
      chunk = w + jj * NW

      @pl.when(chunk < NCHUNKS)
      def _():
        base = chunk * CHUNK
        pltpu.sync_copy(src_hbm.at[pl.ds(base, CHUNK)], src_v)
        pltpu.sync_copy(dst_hbm.at[pl.ds(base, CHUNK)], dst_v)
        pltpu.async_copy(z_hbm.at[src_v], rows_v, sem).wait()
        pltpu.sync_copy(rows_v, agg_sh.at[dst_v], add=True)

      return carry

    lax.fori_loop(0, CHUNKS_PER_W, step, 0)
    plsc.subcore_barrier()
    for k in range(nsub):
      r = pl.ds(row0 + k * CHUNK, CHUNK)
      pltpu.sync_copy(agg_sh.at[r], stage_v)
      pltpu.sync_copy(stage_v, agg_out.at[c].at[r])

  return pl.kernel(
      body, out_type=jax.ShapeDtypeStruct((NC, N_PAD, d), jnp.float32),
      scratch_types=scratch, mesh=_sc_mesh)


def _sc_deg_body(dst_hbm, zdeg_hbm, ones_hbm, deg_out,
                 dst_v, ones_v, stage_v, deg_sh):
  c = lax.axis_index("c")
  s = lax.axis_index("s")
  w = c * NS + s
  row0 = s * ROWS_PER_TILE
  nsub = ROWS_PER_TILE // CHUNK
  pltpu.sync_copy(zdeg_hbm, stage_v)
  for k in range(nsub):
    pltpu.sync_copy(stage_v, deg_sh.at[pl.ds(row0 + k * CHUNK, CHUNK)])
  pltpu.sync_copy(ones_hbm, ones_v)
  plsc.subcore_barrier()

  def step(jj, carry):
    chunk = w + jj * NW

    @pl.when(chunk < NCHUNKS)
    def _():
      pltpu.sync_copy(dst_hbm.at[pl.ds(chunk * CHUNK, CHUNK)], dst_v)
      pltpu.sync_copy(ones_v, deg_sh.at[dst_v], add=True)

    return carry

  lax.fori_loop(0, CHUNKS_PER_W, step, 0)
  plsc.subcore_barrier()
  for k in range(nsub):
    r = pl.ds(row0 + k * CHUNK, CHUNK)
    pltpu.sync_copy(deg_sh.at[r], stage_v)
    pltpu.sync_copy(stage_v, deg_out.at[c].at[r])


_sc_deg = pl.kernel(
    _sc_deg_body,
    out_type=jax.ShapeDtypeStruct((NC, N_PAD, DEG_W), jnp.float32),
    scratch_types=[
        pltpu.VMEM((CHUNK,), jnp.int32),
        pltpu.VMEM((CHUNK, DEG_W), jnp.float32),
        pltpu.VMEM((CHUNK, DEG_W), jnp.float32),
        pltpu.VMEM_SHARED((N_PAD, DEG_W), jnp.float32),
    ],
    mesh=_sc_mesh)

_sc_segsum_128 = _make_sc_segsum(D_HID)

BN = 1000


def _tc_full(shape):
  return pl.BlockSpec(shape, lambda i: (0,) * len(shape))


def _tc_rows(d):
  return pl.BlockSpec((BN, d), lambda i: (i, 0))


def _matmul_body(x_ref, w_ref, o_ref):
  o_ref[...] = jnp.dot(x_ref[...], w_ref[...], precision=lax.Precision.HIGHEST,
                       preferred_element_type=jnp.float32)


def _tc_matmul(x, w):
  d_in, d_o = w.shape
  return pl.pallas_call(
      _matmul_body,
      grid=(N // BN,),
      in_specs=[_tc_rows(d_in), _tc_full(w.shape)],
      out_specs=_tc_rows(d_o),
      out_shape=jax.ShapeDtypeStruct((N, d_o), jnp.float32),
  )(x, w)


def _make_layer_body(with_z):
  def body(h_ref, a0_ref, a1_ref, d0_ref, d1_ref, ws_ref, b_ref,
           wn_ref, h_out_ref, *maybe_z):
    deg = d0_ref[:, :1] + d1_ref[:, :1]
    invd = 1.0 / jnp.maximum(deg, 1.0)
    mean = (a0_ref[...] + a1_ref[...]) * invd
    h_new = jnp.dot(h_ref[...], ws_ref[...], precision=lax.Precision.HIGHEST,
                    preferred_element_type=jnp.float32) + mean + b_ref[...]
    h_new = jnp.maximum(h_new, 0.0)
    h_out_ref[...] = h_new
    if with_z:
      maybe_z[0][...] = jnp.dot(h_new, wn_ref[...], precision=lax.Precision.HIGHEST,
                                preferred_element_type=jnp.float32)
  return body


def _tc_layer(h, agg, deg, w_self, b, w_neigh_next, with_z=True):
  d_n = w_neigh_next.shape[1]
  out_specs = [_tc_rows(D_HID)]
  out_shape = [jax.ShapeDtypeStruct((N, D_HID), jnp.float32)]
  if with_z:
    out_specs.append(_tc_rows(d_n))
    out_shape.append(jax.ShapeDtypeStruct((N, d_n), jnp.float32))
  return pl.pallas_call(
      _make_layer_body(with_z),
      grid=(N // BN,),
      in_specs=[_tc_rows(D_HID), _tc_rows(D_HID), _tc_rows(D_HID),
                _tc_rows(DEG_W), _tc_rows(DEG_W),
                _tc_full(w_self.shape), _tc_full((1, D_HID)),
                _tc_full(w_neigh_next.shape)],
      out_specs=out_specs,
      out_shape=out_shape,
  )(h, agg[0], agg[1], deg[0], deg[1], w_self, b.reshape(1, -1),
    w_neigh_next)


def _final_body(h_ref, a0_ref, a1_ref, d0_ref, d1_ref, ws_ref, wn_ref,
                b_ref, o_ref):
  deg = d0_ref[:, :1] + d1_ref[:, :1]
  invd = 1.0 / jnp.maximum(deg, 1.0)
  mean = (a0_ref[...] + a1_ref[...]) * invd
  o_ref[...] = (jnp.dot(h_ref[...], ws_ref[...], precision=lax.Precision.HIGHEST,
                        preferred_element_type=jnp.float32)
                + jnp.dot(mean, wn_ref[...], precision=lax.Precision.HIGHEST,
                          preferred_element_type=jnp.float32)
                + b_ref[...])


def _tc_final(h, agg, deg, w_self, w_neigh, b):
  return pl.pallas_call(
      _final_body,
      grid=(N // BN,),
      in_specs=[_tc_rows(D_HID), _tc_rows(D_HID), _tc_rows(D_HID),
                _tc_rows(DEG_W), _tc_rows(DEG_W),
                _tc_full(w_self.shape), _tc_full(w_neigh.shape),
                _tc_full((1, D_OUT))],
      out_specs=_tc_rows(D_OUT),
      out_shape=jax.ShapeDtypeStruct((N, D_OUT), jnp.float32),
  )(h, agg[0], agg[1], deg[0], deg[1], w_self, w_neigh, b.reshape(1, -1))


def kernel(x, edge_index, W_self0, W_neigh0, b0, W_self1, W_neigh1, b1,
           W_self2, W_neigh2, b2):
  src = edge_index[0].astype(jnp.int32)
  dst = edge_index[1].astype(jnp.int32)
  zrow128 = jnp.zeros((CHUNK, D_HID), jnp.float32)
  zdeg = jnp.zeros((CHUNK, DEG_W), jnp.float32)
  ones = jnp.ones((CHUNK, DEG_W), jnp.float32)

  z1 = _tc_matmul(x, W_neigh0)
  deg = _sc_deg(dst, zdeg, ones)
  agg1 = _sc_segsum_128(src, dst, z1, zrow128)
  h1, z2 = _tc_layer(x, agg1, deg, W_self0, b0, W_neigh1)
  agg2 = _sc_segsum_128(src, dst, z2, zrow128)
  (h2,) = _tc_layer(h1, agg2, deg, W_self1, b1, W_neigh2, with_z=False)
  agg3 = _sc_segsum_128(src, dst, h2, zrow128)
  return _tc_final(h2, agg3, deg, W_self2, W_neigh2, b2)
